# baseline (speedup 1.0000x reference)
import functools

import jax
import jax.numpy as jnp
from jax import lax
from jax.experimental import pallas as pl
from jax.experimental.pallas import tpu as pltpu
from jax.experimental.pallas import tpu_sc as plsc

NC, NS = 2, 16
NW = NC * NS
B, L, EMB = 16384, 20, 32
LANES = 128
TOT = B * L
BPW = TOT // NW
ROWS_PW = B // NW
RB = 64
NRB = ROWS_PW // RB
C = 1024
NCHUNK = BPW // C
NBUF = 3

VOCAB = 1000000
VB = 512
VMAIN = 999936
NBLK_PW = VMAIN // VB // NW
VLAST = NBLK_PW * NW * VB
VREM = VOCAB - VMAIN

_mesh = plsc.VectorSubcoreMesh(
    core_axis_name="c", subcore_axis_name="s", num_cores=NC, num_subcores=NS
)
_params = pltpu.CompilerParams(
    use_tc_tiling_on_sc=False, needs_layout_passes=False
)


@functools.partial(
    pl.kernel,
    mesh=_mesh,
    out_type=jax.ShapeDtypeStruct((VOCAB, EMB), jnp.float32),
    scratch_types=[
        pltpu.VMEM((2, EMB, VB), jnp.float32),
        pltpu.VMEM((2, VB, EMB), jnp.float32),
        pltpu.SemaphoreType.DMA((2,)),
        pltpu.SemaphoreType.DMA((2,)),
    ],
    compiler_params=_params,
)
def _pack(tmain_hbm, trem_hbm, packed_hbm, src_v, dst_v, lsem, ssem):
    wid = lax.axis_index("s") * NC + lax.axis_index("c")
    lane = lax.iota(jnp.int32, 16)

    def v0_of(i):
        return (i * NW + wid) * VB

    def fire_load(i):
        return pltpu.async_copy(
            tmain_hbm.at[:, pl.ds(v0_of(i), VB)], src_v.at[i % 2], lsem.at[i % 2]
        )

    def transpose_block(s, nrows):
        def tr(r, carry):
            rvec = jnp.full((16,), r, jnp.int32)
            lo = plsc.load_gather(src_v.at[s], [lane, rvec])
            dst_v[s, r, pl.ds(0, 16)] = lo
            hi = plsc.load_gather(src_v.at[s], [lane + 16, rvec])
            dst_v[s, r, pl.ds(16, 16)] = hi
            return carry

        lax.fori_loop(0, nrows, tr, 0)

    def fire_store(i):
        return pltpu.async_copy(
            dst_v.at[i % 2], packed_hbm.at[pl.ds(v0_of(i), VB)], ssem.at[i % 2]
        )

    loads = {0: fire_load(0)}
    stores = {}
    for i in range(NBLK_PW):
        if i + 1 < NBLK_PW:
            loads[i + 1] = fire_load(i + 1)
        loads.pop(i).wait()
        if i >= 2:
            stores.pop(i - 2).wait()
        transpose_block(i % 2, VB)
        stores[i] = fire_store(i)
    for i in sorted(stores):
        stores.pop(i).wait()

    @pl.when(wid == 0)
    def _last_block():
        pltpu.sync_copy(tmain_hbm.at[:, pl.ds(VLAST, VB)], src_v.at[0])
        transpose_block(0, VB)
        pltpu.sync_copy(dst_v.at[0], packed_hbm.at[pl.ds(VLAST, VB)])

    @pl.when(wid == 1)
    def _remainder():
        pltpu.sync_copy(trem_hbm, src_v.at[1, :, pl.ds(0, LANES)])
        transpose_block(1, VREM)
        pltpu.sync_copy(
            dst_v.at[1, pl.ds(0, VREM)], packed_hbm.at[pl.ds(VMAIN, VREM)]
        )


@functools.partial(
    pl.kernel,
    mesh=_mesh,
    out_type=jax.ShapeDtypeStruct((TOT, EMB), jnp.float32),
    scratch_types=[
        pltpu.VMEM((RB, LANES), jnp.int32),
        pltpu.VMEM((BPW + 16,), jnp.int32),
        pltpu.VMEM((NBUF, C, EMB), jnp.float32),
        pltpu.SemaphoreType.DMA((NBUF,)),
        pltpu.SemaphoreType.DMA((NBUF,)),
    ],
    compiler_params=_params,
)
def _gather(y_hbm, table_hbm, out_hbm, yv, idx_v, rows_v, gsem, ssem):
    wid = lax.axis_index("s") * NC + lax.axis_index("c")
    base = wid * BPW
    row_base = wid * ROWS_PW

    lane = lax.iota(jnp.int32, 16)

    def ext_block(b, carry):
        pltpu.sync_copy(y_hbm.at[pl.ds(row_base + b * RB, RB)], yv)

        def ext_row(r, c2):
            rvec = jnp.full((16,), r, jnp.int32)
            dst = (b * RB + r) * L
            lo = plsc.load_gather(yv, [rvec, lane])
            idx_v[pl.ds(dst, 16)] = lo
            hi = plsc.load_gather(yv, [rvec, lane + 16])
            idx_v[pl.ds(dst + 16, 16)] = hi
            return c2

        lax.fori_loop(0, RB, ext_row, 0)
        return carry

    lax.fori_loop(0, NRB, ext_block, 0)

    def fire_gather(c):
        return pltpu.async_copy(
            table_hbm.at[idx_v.at[pl.ds(c * C, C)]],
            rows_v.at[c % NBUF],
            gsem.at[c % NBUF],
        )

    def fire_store(c):
        return pltpu.async_copy(
            rows_v.at[c % NBUF],
            out_hbm.at[pl.ds(base + c * C, C)],
            ssem.at[c % NBUF],
        )

    gathers = {}
    stores = {}
    for c in range(min(2, NCHUNK)):
        gathers[c] = fire_gather(c)
    for c in range(NCHUNK):
        nxt = c + 2
        if nxt < NCHUNK:
            prev = nxt - NBUF
            if prev >= 0:
                stores.pop(prev).wait()
            gathers[nxt] = fire_gather(nxt)
        gathers.pop(c).wait()
        stores[c] = fire_store(c)
    for c in sorted(stores):
        stores.pop(c).wait()


def kernel(y, table):
    tT = table.T
    packed = _pack(
        tT[:, :VMAIN], jnp.pad(tT[:, VMAIN:], ((0, 0), (0, LANES - VREM)))
    )
    y128 = jnp.pad(y, ((0, 0), (0, LANES - L)))
    out = _gather(y128, packed)
    return out.reshape(B, L, EMB)


# speedup vs baseline: 1.0009x; 1.0009x over previous
import functools

import jax
import jax.numpy as jnp
from jax import lax
from jax.experimental import pallas as pl
from jax.experimental.pallas import tpu as pltpu
from jax.experimental.pallas import tpu_sc as plsc

NC, NS = 2, 16
NW = NC * NS
B, L, EMB = 16384, 20, 32
LANES = 128
TOT = B * L
BPW = TOT // NW
ROWS_PW = B // NW
RB = 64
NRB = ROWS_PW // RB
C = 1024
NCHUNK = BPW // C
NBUF = 3

VOCAB = 1000000
VB = 512
VMAIN = 999936
NBLK_PW = VMAIN // VB // NW
VLAST = NBLK_PW * NW * VB
VREM = VOCAB - VMAIN

_mesh = plsc.VectorSubcoreMesh(
    core_axis_name="c", subcore_axis_name="s", num_cores=NC, num_subcores=NS
)
_params = pltpu.CompilerParams(
    use_tc_tiling_on_sc=False, needs_layout_passes=False
)


@functools.partial(
    pl.kernel,
    mesh=_mesh,
    out_type=jax.ShapeDtypeStruct((VOCAB, EMB), jnp.float32),
    scratch_types=[
        pltpu.VMEM((2, EMB, VB), jnp.float32),
        pltpu.VMEM((2, VB, EMB), jnp.float32),
        pltpu.SemaphoreType.DMA((2,)),
        pltpu.SemaphoreType.DMA((2,)),
    ],
    compiler_params=_params,
)
def _pack(tmain_hbm, trem_hbm, packed_hbm, src_v, dst_v, lsem, ssem):
    wid = lax.axis_index("s") * NC + lax.axis_index("c")
    lane = lax.iota(jnp.int32, 16)

    def v0_of(i):
        return (i * NW + wid) * VB

    def fire_load(i):
        return pltpu.async_copy(
            tmain_hbm.at[:, pl.ds(v0_of(i), VB)], src_v.at[i % 2], lsem.at[i % 2]
        )

    def transpose_block(s, nrows, r0=0):
        def tr(r, carry):
            rvec = jnp.full((16,), r, jnp.int32)
            lo = plsc.load_gather(src_v.at[s], [lane, rvec])
            dst_v[s, r, pl.ds(0, 16)] = lo
            hi = plsc.load_gather(src_v.at[s], [lane + 16, rvec])
            dst_v[s, r, pl.ds(16, 16)] = hi
            return carry

        lax.fori_loop(r0, nrows, tr, 0)

    def fire_store(i):
        return pltpu.async_copy(
            dst_v.at[i % 2], packed_hbm.at[pl.ds(v0_of(i), VB)], ssem.at[i % 2]
        )

    loads = {0: fire_load(0)}
    stores = {}
    for i in range(NBLK_PW):
        if i + 1 < NBLK_PW:
            loads[i + 1] = fire_load(i + 1)
        loads.pop(i).wait()
        if i >= 2:
            stores.pop(i - 2).wait()
        transpose_block(i % 2, VB)
        stores[i] = fire_store(i)
    for i in sorted(stores):
        stores.pop(i).wait()

    @pl.when(wid == 0)
    def _last_block():
        pltpu.sync_copy(tmain_hbm.at[:, pl.ds(VLAST, VB)], src_v.at[0])
        transpose_block(0, VB)
        pltpu.sync_copy(dst_v.at[0], packed_hbm.at[pl.ds(VLAST, VB)])

    @pl.when(wid == 1)
    def _remainder():
        pltpu.sync_copy(trem_hbm, src_v.at[1, :, pl.ds(0, LANES)])
        transpose_block(1, LANES, r0=LANES - VREM)
        pltpu.sync_copy(
            dst_v.at[1, pl.ds(LANES - VREM, VREM)],
            packed_hbm.at[pl.ds(VMAIN, VREM)],
        )


@functools.partial(
    pl.kernel,
    mesh=_mesh,
    out_type=jax.ShapeDtypeStruct((TOT, EMB), jnp.float32),
    scratch_types=[
        pltpu.VMEM((RB, LANES), jnp.int32),
        pltpu.VMEM((BPW + 16,), jnp.int32),
        pltpu.VMEM((NBUF, C, EMB), jnp.float32),
        pltpu.SemaphoreType.DMA((NBUF,)),
        pltpu.SemaphoreType.DMA((NBUF,)),
    ],
    compiler_params=_params,
)
def _gather(y_hbm, table_hbm, out_hbm, yv, idx_v, rows_v, gsem, ssem):
    wid = lax.axis_index("s") * NC + lax.axis_index("c")
    base = wid * BPW
    row_base = wid * ROWS_PW

    lane = lax.iota(jnp.int32, 16)

    def ext_block(b, carry):
        pltpu.sync_copy(y_hbm.at[pl.ds(row_base + b * RB, RB)], yv)

        def ext_row(r, c2):
            rvec = jnp.full((16,), r, jnp.int32)
            dst = (b * RB + r) * L
            lo = plsc.load_gather(yv, [rvec, lane])
            idx_v[pl.ds(dst, 16)] = lo
            hi = plsc.load_gather(yv, [rvec, lane + 16])
            idx_v[pl.ds(dst + 16, 16)] = hi
            return c2

        lax.fori_loop(0, RB, ext_row, 0)
        return carry

    lax.fori_loop(0, NRB, ext_block, 0)

    def fire_gather(c):
        return pltpu.async_copy(
            table_hbm.at[idx_v.at[pl.ds(c * C, C)]],
            rows_v.at[c % NBUF],
            gsem.at[c % NBUF],
        )

    def fire_store(c):
        return pltpu.async_copy(
            rows_v.at[c % NBUF],
            out_hbm.at[pl.ds(base + c * C, C)],
            ssem.at[c % NBUF],
        )

    gathers = {}
    stores = {}
    for c in range(min(2, NCHUNK)):
        gathers[c] = fire_gather(c)
    for c in range(NCHUNK):
        nxt = c + 2
        if nxt < NCHUNK:
            prev = nxt - NBUF
            if prev >= 0:
                stores.pop(prev).wait()
            gathers[nxt] = fire_gather(nxt)
        gathers.pop(c).wait()
        stores[c] = fire_store(c)
    for c in sorted(stores):
        stores.pop(c).wait()


def kernel(y, table):
    tT = table.T
    packed = _pack(tT[:, :VMAIN], tT[:, VOCAB - LANES:])
    y128 = jnp.pad(y, ((0, 0), (0, LANES - L)))
    out = _gather(y128, packed)
    return out.reshape(B, L, EMB)


# speedup vs baseline: 4.4986x; 4.4946x over previous
import functools

import jax
import jax.numpy as jnp
from jax import lax
from jax.experimental import pallas as pl
from jax.experimental.pallas import tpu as pltpu
from jax.experimental.pallas import tpu_sc as plsc

NC, NS = 2, 16
NW = NC * NS
B, L, EMB = 16384, 20, 32
LANES = 128
TOT = B * L
BPW = TOT // NW
ROWS_PW = B // NW
RB = 64
NRB = ROWS_PW // RB
C = 1024
NCHUNK = BPW // C
NBUF = 3

_mesh = plsc.VectorSubcoreMesh(
    core_axis_name="c", subcore_axis_name="s", num_cores=NC, num_subcores=NS
)


@functools.partial(
    pl.kernel,
    mesh=_mesh,
    out_type=jax.ShapeDtypeStruct((TOT, EMB), jnp.float32),
    scratch_types=[
        pltpu.VMEM((RB, LANES), jnp.int32),
        pltpu.VMEM((BPW + 16,), jnp.int32),
        pltpu.VMEM((NBUF, C, EMB), jnp.float32),
        pltpu.SemaphoreType.DMA((NBUF,)),
        pltpu.SemaphoreType.DMA((NBUF,)),
    ],
    compiler_params=pltpu.CompilerParams(
        use_tc_tiling_on_sc=False, needs_layout_passes=False
    ),
)
def _gather(y_hbm, table_hbm, out_hbm, yv, idx_v, rows_v, gsem, ssem):
    wid = lax.axis_index("s") * NC + lax.axis_index("c")
    base = wid * BPW
    row_base = wid * ROWS_PW

    lane = lax.iota(jnp.int32, 16)

    def ext_block(b, carry):
        pltpu.sync_copy(y_hbm.at[pl.ds(row_base + b * RB, RB)], yv)

        def ext_row(r, c2):
            rvec = jnp.full((16,), r, jnp.int32)
            dst = (b * RB + r) * L
            lo = plsc.load_gather(yv, [rvec, lane])
            idx_v[pl.ds(dst, 16)] = lo
            hi = plsc.load_gather(yv, [rvec, lane + 16])
            idx_v[pl.ds(dst + 16, 16)] = hi
            return c2

        lax.fori_loop(0, RB, ext_row, 0)
        return carry

    lax.fori_loop(0, NRB, ext_block, 0)

    def fire_gather(c):
        return pltpu.async_copy(
            table_hbm.at[idx_v.at[pl.ds(c * C, C)]],
            rows_v.at[c % NBUF],
            gsem.at[c % NBUF],
        )

    def fire_store(c):
        return pltpu.async_copy(
            rows_v.at[c % NBUF],
            out_hbm.at[pl.ds(base + c * C, C)],
            ssem.at[c % NBUF],
        )

    gathers = {}
    stores = {}
    for c in range(min(2, NCHUNK)):
        gathers[c] = fire_gather(c)
    for c in range(NCHUNK):
        nxt = c + 2
        if nxt < NCHUNK:
            prev = nxt - NBUF
            if prev >= 0:
                stores.pop(prev).wait()
            gathers[nxt] = fire_gather(nxt)
        gathers.pop(c).wait()
        stores[c] = fire_store(c)
    for c in sorted(stores):
        stores.pop(c).wait()


def kernel(y, table):
    y128 = jnp.pad(y, ((0, 0), (0, LANES - L)))
    out = _gather(y128, table)
    flat = lax.optimization_barrier(out.reshape(TOT * EMB))
    return flat.reshape(B, L, EMB)
